# Initial kernel scaffold; baseline (speedup 1.0000x reference)
import jax
import jax.numpy as jnp
from jax.experimental import pallas as pl


def kernel(x, edge_index, W_rel1, b_rel1, W_root1, W_rel2, b_rel2, W_root2, W_out, b_out):
    raise NotImplementedError("write your pallas kernel here")


# speedup vs baseline: 5.2210x; 5.2210x over previous
import jax
import jax.numpy as jnp
from jax import lax
from jax.experimental import pallas as pl
from jax.experimental.pallas import tpu as pltpu
from jax.experimental.pallas import tpu_sc as plsc

N_NODES = 10000
D = 128
DH = D // 2
N_EDGES = 320000

NC = 2
NS = 16
EPS = N_EDGES // NS
K = 80
CHUNKS = EPS // K
ACC_N = 10240
ROWS_PER_SUB = ACC_N // NS
ZR = 16

_MESH = plsc.VectorSubcoreMesh(
    core_axis_name="c", subcore_axis_name="s", num_cores=NC, num_subcores=NS)


def _segsum_body(y_hbm, src_hbm, dst_hbm, out_hbm,
                 src_v, dst_v, rows_v, zbuf, acc, sem):
    cid = lax.axis_index("c")
    sid = lax.axis_index("s")

    zero16 = jnp.zeros((16,), jnp.float32)

    def zrow(i, carry):
        for j in range(DH // 16):
            zbuf[i, pl.ds(j * 16, 16)] = zero16
        return carry

    lax.fori_loop(0, ZR, zrow, 0)
    base = sid * ROWS_PER_SUB
    for t in range(ROWS_PER_SUB // ZR):
        pltpu.sync_copy(zbuf, acc.at[pl.ds(base + t * ZR, ZR)])

    pltpu.sync_copy(src_hbm.at[sid], src_v)
    pltpu.sync_copy(dst_hbm.at[sid], dst_v)
    plsc.subcore_barrier()

    yh = y_hbm.at[cid]

    def step(j, carry):
        pltpu.async_copy(yh.at[src_v.at[j]], rows_v, sem).wait()
        pltpu.sync_copy(rows_v, acc.at[dst_v.at[j]], add=True)
        return carry

    lax.fori_loop(0, CHUNKS, step, 0)
    plsc.subcore_barrier()

    pltpu.sync_copy(acc.at[pl.ds(base, ROWS_PER_SUB)],
                    out_hbm.at[cid, pl.ds(base, ROWS_PER_SUB)])


_segsum = pl.kernel(
    _segsum_body,
    out_type=jax.ShapeDtypeStruct((NC, ACC_N, DH), jnp.float32),
    mesh=_MESH,
    scratch_types=[
        pltpu.VMEM((CHUNKS, K), jnp.int32),
        pltpu.VMEM((CHUNKS, K), jnp.int32),
        pltpu.VMEM((K, DH), jnp.float32),
        pltpu.VMEM((ZR, DH), jnp.float32),
        pltpu.VMEM_SHARED((ACC_N, DH), jnp.float32),
        pltpu.SemaphoreType.DMA,
    ],
    compiler_params=pltpu.CompilerParams(use_tc_tiling_on_sc=False),
)


def _dense1_body(x_ref, wrel_ref, wroot_ref, b_ref, y_ref, r_ref):
    xb = x_ref[...]
    dn = (((1,), (1,)), ((), ()))
    yb = lax.dot_general(xb, wrel_ref[...], dn,
                         preferred_element_type=jnp.float32)
    y_ref[0] = yb[:, :DH]
    y_ref[1] = yb[:, DH:]
    r_ref[...] = lax.dot_general(xb, wroot_ref[...], dn,
                                 preferred_element_type=jnp.float32) + b_ref[...]


def _dense1(x, w_rel, w_root, b, blk):
    n = x.shape[0]
    grid = n // blk
    return pl.pallas_call(
        _dense1_body,
        grid=(grid,),
        in_specs=[
            pl.BlockSpec((blk, D), lambda i: (i, 0)),
            pl.BlockSpec((D, D), lambda i: (0, 0)),
            pl.BlockSpec((D, D), lambda i: (0, 0)),
            pl.BlockSpec((1, D), lambda i: (0, 0)),
        ],
        out_specs=[
            pl.BlockSpec((NC, blk, DH), lambda i: (0, i, 0)),
            pl.BlockSpec((blk, D), lambda i: (i, 0)),
        ],
        out_shape=[
            jax.ShapeDtypeStruct((NC, n, DH), jnp.float32),
            jax.ShapeDtypeStruct((n, D), jnp.float32),
        ],
    )(x, w_rel, w_root, b.reshape(1, D))


def _dense2_body(p_ref, root_ref, wrel_ref, wroot_ref, b_ref, y_ref, r_ref):
    agg = jnp.concatenate([p_ref[0], p_ref[1]], axis=-1)
    h = jnp.maximum(agg + root_ref[...], 0.0)
    dn = (((1,), (1,)), ((), ()))
    yb = lax.dot_general(h, wrel_ref[...], dn,
                         preferred_element_type=jnp.float32)
    y_ref[0] = yb[:, :DH]
    y_ref[1] = yb[:, DH:]
    r_ref[...] = lax.dot_general(h, wroot_ref[...], dn,
                                 preferred_element_type=jnp.float32) + b_ref[...]


def _dense2(p, root, w_rel, w_root, b, blk):
    n = root.shape[0]
    grid = n // blk
    return pl.pallas_call(
        _dense2_body,
        grid=(grid,),
        in_specs=[
            pl.BlockSpec((NC, blk, DH), lambda i: (0, i, 0)),
            pl.BlockSpec((blk, D), lambda i: (i, 0)),
            pl.BlockSpec((D, D), lambda i: (0, 0)),
            pl.BlockSpec((D, D), lambda i: (0, 0)),
            pl.BlockSpec((1, D), lambda i: (0, 0)),
        ],
        out_specs=[
            pl.BlockSpec((NC, blk, DH), lambda i: (0, i, 0)),
            pl.BlockSpec((blk, D), lambda i: (i, 0)),
        ],
        out_shape=[
            jax.ShapeDtypeStruct((NC, n, DH), jnp.float32),
            jax.ShapeDtypeStruct((n, D), jnp.float32),
        ],
    )(p, root, w_rel, w_root, b.reshape(1, D))


def _dense3_body(p_ref, root_ref, wout_ref, bout_ref, o_ref):
    agg = jnp.concatenate([p_ref[0], p_ref[1]], axis=-1)
    h = jnp.maximum(agg + root_ref[...], 0.0)
    o_ref[...] = jnp.sum(h * wout_ref[...], axis=1, keepdims=True) + bout_ref[0, 0]


def _dense3(p, root, w_out, b_out, blk):
    n = root.shape[0]
    grid = n // blk
    return pl.pallas_call(
        _dense3_body,
        grid=(grid,),
        in_specs=[
            pl.BlockSpec((NC, blk, DH), lambda i: (0, i, 0)),
            pl.BlockSpec((blk, D), lambda i: (i, 0)),
            pl.BlockSpec((1, D), lambda i: (0, 0)),
            pl.BlockSpec((1, 1), lambda i: (0, 0)),
        ],
        out_specs=pl.BlockSpec((blk, 1), lambda i: (i, 0)),
        out_shape=jax.ShapeDtypeStruct((n, 1), jnp.float32),
    )(p, root, w_out, b_out.reshape(1, 1))


def kernel(x, edge_index, W_rel1, b_rel1, W_root1, W_rel2, b_rel2, W_root2,
           W_out, b_out):
    src = edge_index[0].astype(jnp.int32).reshape(NS, CHUNKS, K)
    dst = edge_index[1].astype(jnp.int32).reshape(NS, CHUNKS, K)

    blk = 1000
    y1, r1 = _dense1(x, W_rel1, W_root1, b_rel1, blk)
    p1 = _segsum(y1, src, dst)
    y2, r2 = _dense2(p1, r1, W_rel2, W_root2, b_rel2, blk)
    p2 = _segsum(y2, src, dst)
    return _dense3(p2, r2, W_out, b_out, blk)
